# baseline (speedup 1.0000x reference)
import jax
import jax.numpy as jnp
from jax import lax
from jax.experimental import pallas as pl
from jax.experimental.pallas import tpu as pltpu

_N = 2048
_H = 64
_DEPTH = 12


def _gates(z):
    t = jnp.tanh(z)
    f = 0.5 * t[:, :_H] + 0.5
    i = 0.5 * t[:, _H:2 * _H] + 0.5
    u = t[:, 2 * _H:3 * _H]
    o = 0.5 * t[:, 3 * _H:] + 0.5
    return f, i, u, o


def _cell(z, cs):
    f, i, u, o = _gates(z)
    c_new = i * u + f * cs
    h_new = o * jnp.tanh(c_new)
    return h_new, c_new


def _encoder_kernel(adj_ref, x_ref, WgH_ref, WgX_ref, bg_ref,
                    L1W_ref, L1b_ref, L2W_ref, L2b_ref, L3W_ref, L3b_ref,
                    L4W_ref, L4b_ref, y_ref, an_ref, zx_ref):
    adj = adj_ref[0]
    r = jnp.sum(adj, axis=1, keepdims=True)
    inv = 1.0 / (r + 1e-6)
    an_ref[...] = (adj * inv).astype(jnp.bfloat16)
    xs = jnp.dot(an_ref[...], x_ref[0].astype(jnp.bfloat16),
                 preferred_element_type=jnp.float32)
    zx_ref[...] = (jnp.dot(xs, WgX_ref[...], preferred_element_type=jnp.float32)
                   + bg_ref[...]).astype(jnp.bfloat16)

    h, c = _cell(zx_ref[...].astype(jnp.float32), 0.0)
    hc0 = jnp.concatenate([h, c], axis=1).astype(jnp.bfloat16)

    def step(_, hc):
        raw = jnp.dot(an_ref[...], hc, preferred_element_type=jnp.float32)
        z = (jnp.dot(raw[:, :_H].astype(jnp.bfloat16), WgH_ref[...],
                     preferred_element_type=jnp.float32) + zx_ref[...])
        h_new, c_new = _cell(z, raw[:, _H:])
        return jnp.concatenate([h_new, c_new], axis=1).astype(jnp.bfloat16)

    hc = hc0
    for _ in range(_DEPTH - 2):
        hc = step(0, hc)

    raw = jnp.dot(an_ref[0:8, :], hc, preferred_element_type=jnp.float32)
    z = (jnp.dot(raw[:, :_H].astype(jnp.bfloat16), WgH_ref[...],
                 preferred_element_type=jnp.float32) + zx_ref[0:8, :])
    h_new, _ = _cell(z, raw[:, _H:])

    h_root = h_new[0:1, :]
    y1 = jnp.tanh(jnp.dot(h_root, L1W_ref[...],
                          preferred_element_type=jnp.float32) + L1b_ref[...])
    y2 = (jnp.dot(jax.nn.relu(
              jnp.dot(h_root, L2W_ref[...],
                      preferred_element_type=jnp.float32) + L2b_ref[...]),
          L3W_ref[...], preferred_element_type=jnp.float32) + L3b_ref[...])
    y_ref[0] = jax.nn.relu(
        jnp.dot(y1 + y2, L4W_ref[...],
                preferred_element_type=jnp.float32) + L4b_ref[...])


def kernel(node_tensors, adj, W_f, bW_f, b_f, W_i, bW_i, b_i, W_u, bW_u, b_u,
           W_o, bW_o, b_o, L1_W, L1_b, L2_W, L2_b, L3_W, L3_b, L4_W, L4_b):
    B, N, X = node_tensors.shape
    H = W_f.shape[1]
    Wg = jnp.concatenate([0.5 * W_f, 0.5 * W_i, W_u, 0.5 * W_o], axis=1)
    WgH = Wg[:H].astype(jnp.bfloat16)
    WgX = Wg[H:]
    bg = jnp.concatenate([0.5 * (bW_f + b_f), 0.5 * (bW_i + b_i),
                          bW_u + b_u, 0.5 * (bW_o + b_o)]).reshape(1, 4 * H)

    full = lambda shape: pl.BlockSpec(shape, lambda b: (0,) * len(shape))
    return pl.pallas_call(
        _encoder_kernel,
        grid=(B,),
        in_specs=[
            pl.BlockSpec((1, N, N), lambda b: (b, 0, 0)),
            pl.BlockSpec((1, N, X), lambda b: (b, 0, 0)),
            full((H, 4 * H)),
            full((X, 4 * H)),
            full((1, 4 * H)),
            full((H, H)), full((1, H)),
            full((H, H)), full((1, H)),
            full((H, H)), full((1, H)),
            full((H, H)), full((1, H)),
        ],
        out_specs=pl.BlockSpec((1, 1, H), lambda b: (b, 0, 0)),
        out_shape=jax.ShapeDtypeStruct((B, 1, H), jnp.float32),
        scratch_shapes=[
            pltpu.VMEM((N, N), jnp.bfloat16),
            pltpu.VMEM((N, 4 * H), jnp.bfloat16),
        ],
        compiler_params=pltpu.CompilerParams(
            dimension_semantics=("parallel",)),
    )(adj, node_tensors, WgH, WgX, bg,
      L1_W, L1_b.reshape(1, H), L2_W, L2_b.reshape(1, H),
      L3_W, L3_b.reshape(1, H), L4_W, L4_b.reshape(1, H)).reshape(B, H)


# speedup vs baseline: 1.5904x; 1.5904x over previous
import jax
import jax.numpy as jnp
from jax import lax
from jax.experimental import pallas as pl
from jax.experimental.pallas import tpu as pltpu

_N = 2048
_H = 64
_DEPTH = 12


def _gates(z):
    t = jnp.tanh(z)
    f = 0.5 * t[:, :_H] + 0.5
    i = 0.5 * t[:, _H:2 * _H] + 0.5
    u = t[:, 2 * _H:3 * _H]
    o = 0.5 * t[:, 3 * _H:] + 0.5
    return f, i, u, o


def _cell(z, cs):
    f, i, u, o = _gates(z)
    c_new = i * u + f * cs
    h_new = o * jnp.tanh(c_new)
    return h_new, c_new


def _encoder_kernel(adj_ref, x_ref, WgH_ref, WgX_ref, bg_ref,
                    L1W_ref, L1b_ref, L2W_ref, L2b_ref, L3W_ref, L3b_ref,
                    L4W_ref, L4b_ref, y_ref, an_ref, zx_ref, hc_ref):
    adj = adj_ref[0]
    r = jnp.sum(adj, axis=1, keepdims=True)
    inv = 1.0 / (r + 1e-6)
    an = (adj * inv).astype(jnp.bfloat16)
    an_ref[...] = an
    xs = jnp.dot(an, x_ref[0].astype(jnp.bfloat16),
                 preferred_element_type=jnp.float32)
    zx_ref[...] = (jnp.dot(xs, WgX_ref[...], preferred_element_type=jnp.float32)
                   + bg_ref[...])

    h, c = _cell(zx_ref[...], 0.0)
    hc_ref[...] = jnp.concatenate([h, c], axis=1).astype(jnp.bfloat16)

    def step(_, carry):
        raw = jnp.dot(an_ref[...], hc_ref[...],
                      preferred_element_type=jnp.float32)
        z = (jnp.dot(raw[:, :_H], WgH_ref[...],
                     preferred_element_type=jnp.float32) + zx_ref[...])
        h_new, c_new = _cell(z, raw[:, _H:])
        hc_ref[...] = jnp.concatenate([h_new, c_new], axis=1).astype(jnp.bfloat16)
        return carry

    lax.fori_loop(0, _DEPTH - 2, step, 0)

    raw = jnp.dot(an_ref[0:8, :], hc_ref[...], preferred_element_type=jnp.float32)
    z = (jnp.dot(raw[:, :_H], WgH_ref[...], preferred_element_type=jnp.float32)
         + zx_ref[0:8, :])
    h_new, _ = _cell(z, raw[:, _H:])

    h_root = h_new[0:1, :]
    y1 = jnp.tanh(jnp.dot(h_root, L1W_ref[...],
                          preferred_element_type=jnp.float32) + L1b_ref[...])
    y2 = (jnp.dot(jax.nn.relu(
              jnp.dot(h_root, L2W_ref[...],
                      preferred_element_type=jnp.float32) + L2b_ref[...]),
          L3W_ref[...], preferred_element_type=jnp.float32) + L3b_ref[...])
    y_ref[0] = jax.nn.relu(
        jnp.dot(y1 + y2, L4W_ref[...],
                preferred_element_type=jnp.float32) + L4b_ref[...])


def kernel(node_tensors, adj, W_f, bW_f, b_f, W_i, bW_i, b_i, W_u, bW_u, b_u,
           W_o, bW_o, b_o, L1_W, L1_b, L2_W, L2_b, L3_W, L3_b, L4_W, L4_b):
    B, N, X = node_tensors.shape
    H = W_f.shape[1]
    Wg = jnp.concatenate([0.5 * W_f, 0.5 * W_i, W_u, 0.5 * W_o], axis=1)
    WgH = Wg[:H]
    WgX = Wg[H:]
    bg = jnp.concatenate([0.5 * (bW_f + b_f), 0.5 * (bW_i + b_i),
                          bW_u + b_u, 0.5 * (bW_o + b_o)]).reshape(1, 4 * H)

    full = lambda shape: pl.BlockSpec(shape, lambda b: (0,) * len(shape))
    return pl.pallas_call(
        _encoder_kernel,
        grid=(B,),
        in_specs=[
            pl.BlockSpec((1, N, N), lambda b: (b, 0, 0)),
            pl.BlockSpec((1, N, X), lambda b: (b, 0, 0)),
            full((H, 4 * H)),
            full((X, 4 * H)),
            full((1, 4 * H)),
            full((H, H)), full((1, H)),
            full((H, H)), full((1, H)),
            full((H, H)), full((1, H)),
            full((H, H)), full((1, H)),
        ],
        out_specs=pl.BlockSpec((1, 1, H), lambda b: (b, 0, 0)),
        out_shape=jax.ShapeDtypeStruct((B, 1, H), jnp.float32),
        scratch_shapes=[
            pltpu.VMEM((N, N), jnp.bfloat16),
            pltpu.VMEM((N, 4 * H), jnp.float32),
            pltpu.VMEM((N, 2 * H), jnp.bfloat16),
        ],
        compiler_params=pltpu.CompilerParams(
            dimension_semantics=("parallel",)),
    )(adj, node_tensors, WgH, WgX, bg,
      L1_W, L1_b.reshape(1, H), L2_W, L2_b.reshape(1, H),
      L3_W, L3_b.reshape(1, H), L4_W, L4_b.reshape(1, H)).reshape(B, H)
